# baseline (speedup 1.0000x reference)
import functools

import jax
import jax.numpy as jnp
from jax.experimental import pallas as pl
from jax.experimental.pallas import tpu as pltpu

_EPS = 0.001


def _bn_stats(t_ref, n):
    c = t_ref.shape[1]
    half = n // 16

    def fold8(a):
        a4 = a[0:4] + a[4:8]
        a2 = a4[0:2] + a4[2:4]
        return a2[0:1] + a2[1:2]

    def sum_tiles(read_tile):
        def body(i, acc):
            return acc + read_tile(i)
        acc_a = jax.lax.fori_loop(0, half, body,
                                  jnp.zeros((8, c), jnp.float32))
        acc_b = jax.lax.fori_loop(half, 2 * half, body,
                                  jnp.zeros((8, c), jnp.float32))
        return fold8(acc_a) + fold8(acc_b)

    def t_tile(i):
        return t_ref[pl.ds(i * 8, 8), :]

    m = sum_tiles(t_tile) * (1.0 / n)

    def sq_tile(i):
        d = t_tile(i) - m
        return d * d

    v = sum_tiles(sq_tile) * (1.0 / n)
    return m, v


def _lin_bn_kernel(x_ref, w_ref, b_ref, t_ref, o_ref, *, n):
    t_ref[:] = jnp.dot(x_ref[:], w_ref[:],
                       preferred_element_type=jnp.float32) + b_ref[:]
    m, v = _bn_stats(t_ref, n)
    o_ref[:] = (t_ref[:] - m) / jnp.sqrt(v + _EPS)


def _proj_kernel(z_ref, w_ref, o_ref):
    o_ref[:] = jnp.dot(z_ref[:], w_ref[:], preferred_element_type=jnp.float32)


def _gcn1_kernel(adj_ref, y1_ref, b1_ref, w2_ref, y2_ref):
    u = jnp.dot(adj_ref[:], y1_ref[:], preferred_element_type=jnp.float32)
    h = jax.nn.relu(u + b1_ref[:])
    y2_ref[:] = jnp.dot(h, w2_ref[:], preferred_element_type=jnp.float32)


def _gcn2_kernel(adj_ref, y2_ref, b2_ref, out_ref):
    out_ref[:] = jnp.dot(adj_ref[:], y2_ref[:],
                         preferred_element_type=jnp.float32) + b2_ref[:]


def _attdec1_kernel(femb_ref, semb_ref, attw_ref, w1_ref, b1_ref,
                    emb_ref, t_ref, o_ref, *, n):
    f = femb_ref[:]
    s = semb_ref[:]
    aw = attw_ref[:]
    wf = jnp.dot(f, aw, preferred_element_type=jnp.float32)
    ws = jnp.dot(s, aw, preferred_element_type=jnp.float32)
    mx = jnp.maximum(wf, ws)
    ef = jnp.exp(wf - mx)
    es = jnp.exp(ws - mx)
    denom = ef + es
    bf = ef / denom
    bs = es / denom
    emb = bf * f + bs * s
    emb_ref[:] = emb
    t_ref[:] = jnp.dot(emb, w1_ref[:],
                       preferred_element_type=jnp.float32) + b1_ref[:]
    m, v = _bn_stats(t_ref, n)
    o_ref[:] = (t_ref[:] - m) / jnp.sqrt(v + _EPS)


def _dec2_kernel(h_ref, w2_ref, b2_ref, t_ref, o_ref, *, n):
    t_ref[:] = jnp.dot(h_ref[:], w2_ref[:],
                       preferred_element_type=jnp.float32) + b2_ref[:]
    m, v = _bn_stats(t_ref, n)
    o_ref[:] = jax.nn.sigmoid((t_ref[:] - m) / jnp.sqrt(v + _EPS))


def _lin_bn(x, w, b):
    n = x.shape[0]
    c = w.shape[1]
    kern = functools.partial(_lin_bn_kernel, n=n)
    _, o = pl.pallas_call(
        kern,
        out_shape=[jax.ShapeDtypeStruct((n, c), jnp.float32),
                   jax.ShapeDtypeStruct((n, c), jnp.float32)],
    )(x, w, b.reshape(1, -1))
    return o


def _run_gcn(adj, y1, gc1_b, gc2_W, gc2_b, *, bm):
    n, nemb = y1.shape
    nblk = n // bm
    y2 = pl.pallas_call(
        _gcn1_kernel,
        grid=(nblk,),
        in_specs=[
            pl.BlockSpec((bm, n), lambda i: (i, 0)),
            pl.BlockSpec((n, nemb), lambda i: (0, 0)),
            pl.BlockSpec((1, nemb), lambda i: (0, 0)),
            pl.BlockSpec((nemb, nemb), lambda i: (0, 0)),
        ],
        out_specs=pl.BlockSpec((bm, nemb), lambda i: (i, 0)),
        out_shape=jax.ShapeDtypeStruct((n, nemb), jnp.float32),
    )(adj, y1, gc1_b, gc2_W)
    return pl.pallas_call(
        _gcn2_kernel,
        grid=(nblk,),
        in_specs=[
            pl.BlockSpec((bm, n), lambda i: (i, 0)),
            pl.BlockSpec((n, nemb), lambda i: (0, 0)),
            pl.BlockSpec((1, nemb), lambda i: (0, 0)),
        ],
        out_specs=pl.BlockSpec((bm, nemb), lambda i: (i, 0)),
        out_shape=jax.ShapeDtypeStruct((n, nemb), jnp.float32),
    )(adj, y2, gc2_b)


def kernel(feat, fadj, sadj, enc_W0, enc_b0, enc_W1, enc_b1, gc1_W, gc1_b,
           gc2_W, gc2_b, att_W, dec_W1, dec_b1, dec_W2, dec_b2):
    n = feat.shape[0]
    nemb = gc1_W.shape[1]

    z = jax.nn.elu(_lin_bn(feat, enc_W0, enc_b0))
    z = jax.nn.elu(_lin_bn(z, enc_W1, enc_b1))

    y1 = pl.pallas_call(
        _proj_kernel,
        out_shape=jax.ShapeDtypeStruct((n, nemb), jnp.float32),
    )(z, gc1_W)

    bm = 400
    femb = _run_gcn(fadj, y1, gc1_b.reshape(1, -1), gc2_W,
                    gc2_b.reshape(1, -1), bm=bm)
    semb = _run_gcn(sadj, y1, gc1_b.reshape(1, -1), gc2_W,
                    gc2_b.reshape(1, -1), bm=bm)

    c1 = dec_W1.shape[1]
    emb, _, x3 = pl.pallas_call(
        functools.partial(_attdec1_kernel, n=n),
        out_shape=[
            jax.ShapeDtypeStruct((n, nemb), jnp.float32),
            jax.ShapeDtypeStruct((n, c1), jnp.float32),
            jax.ShapeDtypeStruct((n, c1), jnp.float32),
        ],
    )(femb, semb, att_W, dec_W1, dec_b1.reshape(1, -1))

    hdec = jax.nn.elu(x3)

    c2 = dec_W2.shape[1]
    _, de = pl.pallas_call(
        functools.partial(_dec2_kernel, n=n),
        out_shape=[
            jax.ShapeDtypeStruct((n, c2), jnp.float32),
            jax.ShapeDtypeStruct((n, c2), jnp.float32),
        ],
    )(hdec, dec_W2, dec_b2.reshape(1, -1))

    return femb, semb, de, emb
